# baseline (speedup 1.0000x reference)
import functools

import jax
import jax.numpy as jnp
from jax import lax
from jax.experimental import pallas as pl
from jax.experimental.pallas import tpu as pltpu
from jax.experimental.pallas import tpu_sc as plsc

K = 20
KP = 20
BQ = 256
CP = 16
NSUB = 128
_SUBS = [(0, 128), (128, 128), (256, 64)]


def _prep_body(x_ref, w1_ref, wd_ref, bias_ref, y_ref, z_ref, sq_ref):
    x = x_ref[0]
    y_ref[0] = jnp.dot(x, w1_ref[...], preferred_element_type=jnp.float32)
    z_ref[0] = (
        jnp.dot(x, wd_ref[...], preferred_element_type=jnp.float32) + bias_ref[...]
    )
    sq_ref[0] = jnp.sum(x * x, axis=1, keepdims=True)


def _topk_body(b0, sqf_ref, xr_ref, xf_ref, idx_ref, d_ref, iota_ref):
    b = pl.program_id(0)
    xr = xr_ref[0]
    xf = xf_ref[0]
    n = xf.shape[0]
    bq = xr.shape[0]
    @pl.when((pl.program_id(0) == 0) & (pl.program_id(1) == 0))
    def _():
        iota_ref[...] = lax.broadcasted_iota(jnp.int32, (n, bq), 0)

    d_ref[...] = sqf_ref[0] - 2.0 * lax.dot_general(
        xf, xr, (((1,), (1,)), ((), ())), preferred_element_type=jnp.float32
    )

    nv = n // 8
    nacc = 4
    blk = nv // nacc
    row_iota = lax.broadcasted_iota(jnp.int32, (KP, bq), 0)
    big = jnp.full((8, bq), n, jnp.int32)
    inf8 = jnp.full((8, bq), jnp.inf, jnp.float32)

    def body(t, carry):
        jp, idxacc = carry
        jpb = jnp.broadcast_to(jp, (8, bq))
        mvals = [inf8] * nacc
        midxs = [big] * nacc
        for a in range(nacc):
            mv, mi = mvals[a], midxs[a]
            for r in range(a * blk, (a + 1) * blk):
                sl = pl.ds(8 * r, 8)
                v = d_ref[sl, :]
                idxr = iota_ref[sl, :]
                v = jnp.where(idxr == jpb, jnp.inf, v)
                d_ref[sl, :] = v
                c = v < mv
                mv = jnp.where(c, v, mv)
                mi = jnp.where(c, idxr, mi)
            mvals[a], midxs[a] = mv, mi
        step = 1
        while step < nacc:
            for a in range(0, nacc, 2 * step):
                c = mvals[a + step] < mvals[a]
                mvals[a] = jnp.where(c, mvals[a + step], mvals[a])
                midxs[a] = jnp.where(c, midxs[a + step], midxs[a])
            step *= 2
        mval8, midx8 = mvals[0], midxs[0]
        m = jnp.min(mval8, axis=0, keepdims=True)
        jmin = jnp.min(jnp.where(mval8 == m, midx8, n), axis=0, keepdims=True)
        idxacc = jnp.where(row_iota >= t, jmin, idxacc)
        return jmin, idxacc

    jp0 = jnp.full((1, bq), -1, jnp.int32)
    idx0 = jnp.zeros((KP, bq), jnp.int32)
    _, idxacc = lax.fori_loop(0, K, body, (jp0, idx0))
    idx_ref[0] = idxacc + (b + b0) * n


def _sc_gather_max(P, C):
    NC, NS = 2, 16
    NW = NC * NS
    PPW = P // NW
    NCH = PPW // CP
    assert sum(s for _, s in _SUBS) == CP * KP

    mesh = plsc.VectorSubcoreMesh(core_axis_name="c", subcore_axis_name="s")

    @functools.partial(
        pl.kernel,
        mesh=mesh,
        out_type=jax.ShapeDtypeStruct((P, C), jnp.float32),
        scratch_types=[
            pltpu.VMEM((CP * KP,), jnp.int32),
            pltpu.VMEM((CP * KP,), jnp.int32),
            pltpu.VMEM((CP * KP, C), jnp.float32),
            pltpu.VMEM((CP * KP, C), jnp.float32),
            pltpu.VMEM((CP, C), jnp.float32),
            pltpu.VMEM((CP, C), jnp.float32),
            pltpu.VMEM((CP, C), jnp.float32),
            pltpu.SemaphoreType.DMA,
            pltpu.SemaphoreType.DMA,
            pltpu.SemaphoreType.DMA,
            pltpu.SemaphoreType.DMA,
        ],
    )
    def gather_max(y_hbm, idx_hbm, z_hbm, out_hbm,
                   idx0, idx1, rows0, rows1, z0, z1, out_v,
                   gsem0, gsem1, zsem0, zsem1):
        c = lax.axis_index("c")
        s = lax.axis_index("s")
        wid = s * NC + c
        base = wid * PPW

        def prefetch(g, idxb, rowsb, zb, gsem, zsem):
            pbase = base + g * CP
            off = pl.multiple_of(pbase * KP, 8)
            pltpu.sync_copy(idx_hbm.at[pl.ds(off, CP * KP)], idxb)
            for so, sz in _SUBS:
                pltpu.make_async_copy(
                    y_hbm.at[idxb.at[pl.ds(so, sz)]],
                    rowsb.at[pl.ds(so, sz)],
                    gsem,
                ).start()
            pltpu.make_async_copy(z_hbm.at[pl.ds(pbase, CP)], zb, zsem).start()

        def wait_chunk(g, idxb, rowsb, zb, gsem, zsem):
            pbase = base + g * CP
            for so, sz in _SUBS:
                pltpu.make_async_copy(
                    y_hbm.at[idxb.at[pl.ds(so, sz)]],
                    rowsb.at[pl.ds(so, sz)],
                    gsem,
                ).wait()
            pltpu.make_async_copy(z_hbm.at[pl.ds(pbase, CP)], zb, zsem).wait()

        def compute(g, rowsb, zb):
            pbase = base + g * CP

            def pt(p, _):
                rbase = p * KP
                for cg in range(C // 16):
                    sl = pl.ds(cg * 16, 16)
                    acc = rowsb[rbase, sl]
                    for j in range(1, KP):
                        acc = jnp.maximum(acc, rowsb[rbase + j, sl])
                    h = acc + zb[p, sl]
                    out_v[p, sl] = jnp.where(h > 0, h, 0.2 * h)
                return 0

            lax.fori_loop(0, CP, pt, 0)
            pltpu.sync_copy(out_v, out_hbm.at[pl.ds(pbase, CP)])

        prefetch(0, idx0, rows0, z0, gsem0, zsem0)

        def super_body(hh, _):
            g0 = 2 * hh
            prefetch(g0 + 1, idx1, rows1, z1, gsem1, zsem1)
            wait_chunk(g0, idx0, rows0, z0, gsem0, zsem0)
            compute(g0, rows0, z0)

            @pl.when(hh < NCH // 2 - 1)
            def _():
                prefetch(g0 + 2, idx0, rows0, z0, gsem0, zsem0)

            wait_chunk(g0 + 1, idx1, rows1, z1, gsem1, zsem1)
            compute(g0 + 1, rows1, z1)
            return 0

        lax.fori_loop(0, NCH // 2, super_body, 0)

    return gather_max


def kernel(x, W, b):
    B, N, C = x.shape
    w1 = W[:C]
    wd = W[C:] - W[:C]

    y, z, sq = pl.pallas_call(
        _prep_body,
        grid=(B,),
        in_specs=[
            pl.BlockSpec((1, N, C), lambda i: (i, 0, 0)),
            pl.BlockSpec((C, C), lambda i: (0, 0)),
            pl.BlockSpec((C, C), lambda i: (0, 0)),
            pl.BlockSpec((C,), lambda i: (0,)),
        ],
        out_specs=[
            pl.BlockSpec((1, N, C), lambda i: (i, 0, 0)),
            pl.BlockSpec((1, N, C), lambda i: (i, 0, 0)),
            pl.BlockSpec((1, N, 1), lambda i: (i, 0, 0)),
        ],
        out_shape=[
            jax.ShapeDtypeStruct((B, N, C), jnp.float32),
            jax.ShapeDtypeStruct((B, N, C), jnp.float32),
            jax.ShapeDtypeStruct((B, N, 1), jnp.float32),
        ],
    )(x, w1, wd, b)

    P = B * N
    NH = 2
    BH = B // NH
    yf = y.reshape(P, C)
    sc_call = _sc_gather_max(BH * N, C)
    outs = []
    for h in range(NH):
        xh = x[h * BH:(h + 1) * BH]
        sqh = sq[h * BH:(h + 1) * BH]
        idx_t = pl.pallas_call(
            functools.partial(_topk_body, h * BH),
            grid=(BH, N // BQ),
            in_specs=[
                pl.BlockSpec((1, N, 1), lambda i, r: (i, 0, 0)),
                pl.BlockSpec((1, BQ, C), lambda i, r: (i, r, 0)),
                pl.BlockSpec((1, N, C), lambda i, r: (i, 0, 0)),
            ],
            out_specs=pl.BlockSpec((1, KP, BQ), lambda i, r: (i, 0, r)),
            out_shape=jax.ShapeDtypeStruct((BH, KP, N), jnp.int32),
            scratch_shapes=[
                pltpu.VMEM((N, BQ), jnp.float32),
                pltpu.VMEM((N, BQ), jnp.int32),
            ],
        )(sqh, xh, xh)
        idx = jnp.transpose(idx_t, (0, 2, 1))
        zh = z[h * BH:(h + 1) * BH]
        outs.append(sc_call(yf, idx.reshape(BH * N * KP),
                            zh.reshape(BH * N, C)))
    out = jnp.concatenate(outs, axis=0)
    return out.reshape(B, N, C)


# speedup vs baseline: 1.0571x; 1.0571x over previous
import functools

import jax
import jax.numpy as jnp
from jax import lax
from jax.experimental import pallas as pl
from jax.experimental.pallas import tpu as pltpu
from jax.experimental.pallas import tpu_sc as plsc

K = 20
KP = 20
BQ = 256
CP = 16
NSUB = 128
_SUBS = [(0, 128), (128, 128), (256, 64)]


def _prep_body(x_ref, w1_ref, wd_ref, bias_ref, y_ref, z_ref, sq_ref):
    x = x_ref[0]
    y_ref[0] = jnp.dot(x, w1_ref[...], preferred_element_type=jnp.float32)
    z_ref[0] = (
        jnp.dot(x, wd_ref[...], preferred_element_type=jnp.float32) + bias_ref[...]
    )
    sq_ref[0] = jnp.sum(x * x, axis=1, keepdims=True)


def _topk_body(b0, sqf_ref, xr_ref, xf_ref, idx_ref, d_ref, iota_ref):
    b = pl.program_id(0)
    xr = xr_ref[0]
    xf = xf_ref[0]
    n = xf.shape[0]
    bq = xr.shape[0]
    @pl.when((pl.program_id(0) == 0) & (pl.program_id(1) == 0))
    def _():
        iota_ref[...] = lax.broadcasted_iota(jnp.int32, (n, bq), 0)

    d_ref[...] = lax.dot_general(
        xf, xr * -2.0, (((1,), (1,)), ((), ())),
        preferred_element_type=jnp.float32,
    )

    nv = n // 8
    nacc = 4
    blk = nv // nacc
    row_iota = lax.broadcasted_iota(jnp.int32, (KP, bq), 0)
    big = jnp.full((8, bq), n, jnp.int32)
    inf8 = jnp.full((8, bq), jnp.inf, jnp.float32)
    sqv = sqf_ref[0]

    def combine_and_pick(mvals, midxs):
        step = 1
        while step < nacc:
            for a in range(0, nacc, 2 * step):
                c = mvals[a + step] < mvals[a]
                mvals[a] = jnp.where(c, mvals[a + step], mvals[a])
                midxs[a] = jnp.where(c, midxs[a + step], midxs[a])
            step *= 2
        mval8, midx8 = mvals[0], midxs[0]
        m = jnp.min(mval8, axis=0, keepdims=True)
        return jnp.min(jnp.where(mval8 == m, midx8, n), axis=0, keepdims=True)

    mvals = [inf8] * nacc
    midxs = [big] * nacc
    for a in range(nacc):
        mv, mi = mvals[a], midxs[a]
        for r in range(a * blk, (a + 1) * blk):
            sl = pl.ds(8 * r, 8)
            v = d_ref[sl, :] + jnp.broadcast_to(sqv[8 * r:8 * r + 8], (8, bq))
            d_ref[sl, :] = v
            idxr = iota_ref[sl, :]
            c = v < mv
            mv = jnp.where(c, v, mv)
            mi = jnp.where(c, idxr, mi)
        mvals[a], midxs[a] = mv, mi
    jmin0 = combine_and_pick(mvals, midxs)

    def body(t, carry):
        jp, idxacc = carry
        jpb = jnp.broadcast_to(jp, (8, bq))
        mvals = [inf8] * nacc
        midxs = [big] * nacc
        for a in range(nacc):
            mv, mi = mvals[a], midxs[a]
            for r in range(a * blk, (a + 1) * blk):
                sl = pl.ds(8 * r, 8)
                v = d_ref[sl, :]
                idxr = iota_ref[sl, :]
                v = jnp.where(idxr == jpb, jnp.inf, v)
                d_ref[sl, :] = v
                c = v < mv
                mv = jnp.where(c, v, mv)
                mi = jnp.where(c, idxr, mi)
            mvals[a], midxs[a] = mv, mi
        jmin = combine_and_pick(mvals, midxs)
        idxacc = jnp.where(row_iota >= t, jmin, idxacc)
        return jmin, idxacc

    idx0 = jnp.broadcast_to(jmin0, (KP, bq))
    _, idxacc = lax.fori_loop(1, K, body, (jmin0, idx0))
    idx_ref[0] = idxacc + (b + b0) * n


def _sc_gather_max(P, C):
    NC, NS = 2, 16
    NW = NC * NS
    PPW = P // NW
    NCH = PPW // CP
    assert sum(s for _, s in _SUBS) == CP * KP

    mesh = plsc.VectorSubcoreMesh(core_axis_name="c", subcore_axis_name="s")

    @functools.partial(
        pl.kernel,
        mesh=mesh,
        out_type=jax.ShapeDtypeStruct((P, C), jnp.float32),
        scratch_types=[
            pltpu.VMEM((CP * KP,), jnp.int32),
            pltpu.VMEM((CP * KP,), jnp.int32),
            pltpu.VMEM((CP * KP, C), jnp.float32),
            pltpu.VMEM((CP * KP, C), jnp.float32),
            pltpu.VMEM((CP, C), jnp.float32),
            pltpu.VMEM((CP, C), jnp.float32),
            pltpu.VMEM((CP, C), jnp.float32),
            pltpu.SemaphoreType.DMA,
            pltpu.SemaphoreType.DMA,
            pltpu.SemaphoreType.DMA,
            pltpu.SemaphoreType.DMA,
        ],
    )
    def gather_max(y_hbm, idx_hbm, z_hbm, out_hbm,
                   idx0, idx1, rows0, rows1, z0, z1, out_v,
                   gsem0, gsem1, zsem0, zsem1):
        c = lax.axis_index("c")
        s = lax.axis_index("s")
        wid = s * NC + c
        base = wid * PPW

        def prefetch(g, idxb, rowsb, zb, gsem, zsem):
            pbase = base + g * CP
            off = pl.multiple_of(pbase * KP, 8)
            pltpu.sync_copy(idx_hbm.at[pl.ds(off, CP * KP)], idxb)
            for so, sz in _SUBS:
                pltpu.make_async_copy(
                    y_hbm.at[idxb.at[pl.ds(so, sz)]],
                    rowsb.at[pl.ds(so, sz)],
                    gsem,
                ).start()
            pltpu.make_async_copy(z_hbm.at[pl.ds(pbase, CP)], zb, zsem).start()

        def wait_chunk(g, idxb, rowsb, zb, gsem, zsem):
            pbase = base + g * CP
            for so, sz in _SUBS:
                pltpu.make_async_copy(
                    y_hbm.at[idxb.at[pl.ds(so, sz)]],
                    rowsb.at[pl.ds(so, sz)],
                    gsem,
                ).wait()
            pltpu.make_async_copy(z_hbm.at[pl.ds(pbase, CP)], zb, zsem).wait()

        def compute(g, rowsb, zb):
            pbase = base + g * CP

            def pt(p, _):
                rbase = p * KP
                for cg in range(C // 16):
                    sl = pl.ds(cg * 16, 16)
                    acc = rowsb[rbase, sl]
                    for j in range(1, KP):
                        acc = jnp.maximum(acc, rowsb[rbase + j, sl])
                    h = acc + zb[p, sl]
                    out_v[p, sl] = jnp.where(h > 0, h, 0.2 * h)
                return 0

            lax.fori_loop(0, CP, pt, 0)
            pltpu.sync_copy(out_v, out_hbm.at[pl.ds(pbase, CP)])

        prefetch(0, idx0, rows0, z0, gsem0, zsem0)

        def super_body(hh, _):
            g0 = 2 * hh
            prefetch(g0 + 1, idx1, rows1, z1, gsem1, zsem1)
            wait_chunk(g0, idx0, rows0, z0, gsem0, zsem0)
            compute(g0, rows0, z0)

            @pl.when(hh < NCH // 2 - 1)
            def _():
                prefetch(g0 + 2, idx0, rows0, z0, gsem0, zsem0)

            wait_chunk(g0 + 1, idx1, rows1, z1, gsem1, zsem1)
            compute(g0 + 1, rows1, z1)
            return 0

        lax.fori_loop(0, NCH // 2, super_body, 0)

    return gather_max


def kernel(x, W, b):
    B, N, C = x.shape
    w1 = W[:C]
    wd = W[C:] - W[:C]

    y, z, sq = pl.pallas_call(
        _prep_body,
        grid=(B,),
        in_specs=[
            pl.BlockSpec((1, N, C), lambda i: (i, 0, 0)),
            pl.BlockSpec((C, C), lambda i: (0, 0)),
            pl.BlockSpec((C, C), lambda i: (0, 0)),
            pl.BlockSpec((C,), lambda i: (0,)),
        ],
        out_specs=[
            pl.BlockSpec((1, N, C), lambda i: (i, 0, 0)),
            pl.BlockSpec((1, N, C), lambda i: (i, 0, 0)),
            pl.BlockSpec((1, N, 1), lambda i: (i, 0, 0)),
        ],
        out_shape=[
            jax.ShapeDtypeStruct((B, N, C), jnp.float32),
            jax.ShapeDtypeStruct((B, N, C), jnp.float32),
            jax.ShapeDtypeStruct((B, N, 1), jnp.float32),
        ],
    )(x, w1, wd, b)

    P = B * N
    NH = 4
    BH = B // NH
    yf = y.reshape(P, C)
    sc_call = _sc_gather_max(BH * N, C)
    outs = []
    for h in range(NH):
        xh = x[h * BH:(h + 1) * BH]
        sqh = sq[h * BH:(h + 1) * BH]
        idx_t = pl.pallas_call(
            functools.partial(_topk_body, h * BH),
            grid=(BH, N // BQ),
            in_specs=[
                pl.BlockSpec((1, N, 1), lambda i, r: (i, 0, 0)),
                pl.BlockSpec((1, BQ, C), lambda i, r: (i, r, 0)),
                pl.BlockSpec((1, N, C), lambda i, r: (i, 0, 0)),
            ],
            out_specs=pl.BlockSpec((1, KP, BQ), lambda i, r: (i, 0, r)),
            out_shape=jax.ShapeDtypeStruct((BH, KP, N), jnp.int32),
            scratch_shapes=[
                pltpu.VMEM((N, BQ), jnp.float32),
                pltpu.VMEM((N, BQ), jnp.int32),
            ],
        )(sqh, xh, xh)
        idx = jnp.transpose(idx_t, (0, 2, 1))
        zh = z[h * BH:(h + 1) * BH]
        outs.append(sc_call(yf, idx.reshape(BH * N * KP),
                            zh.reshape(BH * N, C)))
    out = jnp.concatenate(outs, axis=0)
    return out.reshape(B, N, C)
